# Initial kernel scaffold; baseline (speedup 1.0000x reference)
import jax
import jax.numpy as jnp
from jax.experimental import pallas as pl


def kernel(i_w, i_c, words, contexts):
    raise NotImplementedError("write your pallas kernel here")


# speedup vs baseline: 2.3589x; 2.3589x over previous
import functools

import jax
import jax.numpy as jnp
from jax import lax
from jax.experimental import pallas as pl
from jax.experimental.pallas import tpu as pltpu
from jax.experimental.pallas import tpu_sc as plsc

_DIM = 50
_DIMP = 64
_BATCH = 16384
_NC = 2
_NS = 16
_L = 16
_NW = _NC * _NS
_BPW = _BATCH // _NW
_CH = 128
_NCH = _BPW // _CH

_mesh = plsc.VectorSubcoreMesh(core_axis_name="c", subcore_axis_name="s")


@functools.partial(
    pl.kernel,
    out_type=jax.ShapeDtypeStruct((_BATCH,), jnp.float32),
    mesh=_mesh,
    scratch_types=[
        pltpu.VMEM((_NCH, _CH), jnp.int32),
        pltpu.VMEM((_NCH, _CH), jnp.int32),
        pltpu.VMEM((_BPW, _DIMP), jnp.float32),
        pltpu.VMEM((_BPW, _DIMP), jnp.float32),
        pltpu.VMEM((_BPW,), jnp.float32),
        pltpu.SemaphoreType.DMA,
    ],
    compiler_params=pltpu.CompilerParams(
        use_tc_tiling_on_sc=False, needs_layout_passes=False),
)
def _sc_dot(iw_hbm, ic_hbm, words_hbm, ctx_hbm, out_hbm,
            iw_v, ic_v, wrows, crows, outv, sem):
    wid = lax.axis_index("s") * _NC + lax.axis_index("c")
    rbase = wid * _NCH

    pltpu.sync_copy(iw_hbm.at[pl.ds(rbase, _NCH)], iw_v)
    pltpu.sync_copy(ic_hbm.at[pl.ds(rbase, _NCH)], ic_v)

    copies = []
    for j in range(_NCH):
        dst = pl.ds(j * _CH, _CH)
        copies.append(pltpu.async_copy(words_hbm.at[iw_v.at[j]], wrows.at[dst], sem))
        copies.append(pltpu.async_copy(ctx_hbm.at[ic_v.at[j]], crows.at[dst], sem))
    for cp in copies:
        cp.wait()

    def group(g, carry):
        rows = g * _L + lax.iota(jnp.int32, _L)
        acc = jnp.zeros((_L,), jnp.float32)
        for d in range(_DIM):
            col = jnp.full((_L,), d, jnp.int32)
            w = plsc.load_gather(wrows, [rows, col])
            c = plsc.load_gather(crows, [rows, col])
            acc = acc + w * c
        outv[pl.ds(g * _L, _L)] = acc
        return carry

    lax.fori_loop(0, _BPW // _L, group, 0)

    pltpu.sync_copy(outv, out_hbm.at[pl.ds(wid * _BPW, _BPW)])


def kernel(i_w, i_c, words, contexts):
    iw2 = i_w.astype(jnp.int32).reshape(_BATCH // _CH, _CH)
    ic2 = i_c.astype(jnp.int32).reshape(_BATCH // _CH, _CH)
    wp = jnp.pad(words, ((0, 0), (0, _DIMP - _DIM)))
    cp = jnp.pad(contexts, ((0, 0), (0, _DIMP - _DIM)))
    out = _sc_dot(iw2, ic2, wp, cp)
    return out.reshape(_BATCH, 1, 1)


# speedup vs baseline: 3.7299x; 1.5813x over previous
import functools

import jax
import jax.numpy as jnp
from jax import lax
from jax.experimental import pallas as pl
from jax.experimental.pallas import tpu as pltpu
from jax.experimental.pallas import tpu_sc as plsc

_DIM = 50
_BATCH = 16384
_NC = 2
_NS = 16
_L = 16
_NW = _NC * _NS
_BPW = _BATCH // _NW
_CH = 128
_NCH = _BPW // _CH

_mesh = plsc.VectorSubcoreMesh(core_axis_name="c", subcore_axis_name="s")


@functools.partial(
    pl.kernel,
    out_type=jax.ShapeDtypeStruct((_BATCH,), jnp.float32),
    mesh=_mesh,
    scratch_types=[
        pltpu.VMEM((_NCH, _CH), jnp.int32),
        pltpu.VMEM((_NCH, _CH), jnp.int32),
        pltpu.VMEM((_BPW, _DIM), jnp.float32),
        pltpu.VMEM((_BPW, _DIM), jnp.float32),
        pltpu.VMEM((_BPW,), jnp.float32),
        pltpu.SemaphoreType.DMA,
    ],
    compiler_params=pltpu.CompilerParams(
        use_tc_tiling_on_sc=False, needs_layout_passes=False),
)
def _sc_dot(iw_hbm, ic_hbm, words_hbm, ctx_hbm, out_hbm,
            iw_v, ic_v, wrows, crows, outv, sem):
    wid = lax.axis_index("s") * _NC + lax.axis_index("c")
    rbase = wid * _NCH

    pltpu.sync_copy(iw_hbm.at[pl.ds(rbase, _NCH)], iw_v)
    pltpu.sync_copy(ic_hbm.at[pl.ds(rbase, _NCH)], ic_v)

    copies = []
    for j in range(_NCH):
        dst = pl.ds(j * _CH, _CH)
        copies.append(pltpu.async_copy(words_hbm.at[iw_v.at[j]], wrows.at[dst], sem))
        copies.append(pltpu.async_copy(ctx_hbm.at[ic_v.at[j]], crows.at[dst], sem))
    for cp in copies:
        cp.wait()

    def group(g, carry):
        rows = g * _L + lax.iota(jnp.int32, _L)
        acc = jnp.zeros((_L,), jnp.float32)
        for d in range(_DIM):
            col = jnp.full((_L,), d, jnp.int32)
            w = plsc.load_gather(wrows, [rows, col])
            c = plsc.load_gather(crows, [rows, col])
            acc = acc + w * c
        outv[pl.ds(g * _L, _L)] = acc
        return carry

    lax.fori_loop(0, _BPW // _L, group, 0)

    pltpu.sync_copy(outv, out_hbm.at[pl.ds(wid * _BPW, _BPW)])


def kernel(i_w, i_c, words, contexts):
    iw2 = i_w.astype(jnp.int32).reshape(_BATCH // _CH, _CH)
    ic2 = i_c.astype(jnp.int32).reshape(_BATCH // _CH, _CH)
    out = _sc_dot(iw2, ic2, words, contexts)
    return out.reshape(_BATCH, 1, 1)


# speedup vs baseline: 3.7481x; 1.0049x over previous
import functools

import jax
import jax.numpy as jnp
from jax import lax
from jax.experimental import pallas as pl
from jax.experimental.pallas import tpu as pltpu
from jax.experimental.pallas import tpu_sc as plsc

_DIM = 50
_DIMP = 56
_BATCH = 16384
_NC = 2
_NS = 16
_L = 16
_NW = _NC * _NS
_BPW = _BATCH // _NW
_CH = 128
_NCH = _BPW // _CH

_mesh = plsc.VectorSubcoreMesh(core_axis_name="c", subcore_axis_name="s")


@functools.partial(
    pl.kernel,
    out_type=jax.ShapeDtypeStruct((_BATCH,), jnp.float32),
    mesh=_mesh,
    scratch_types=[
        pltpu.VMEM((_NCH, _CH), jnp.int32),
        pltpu.VMEM((_NCH, _CH), jnp.int32),
        pltpu.VMEM((_BPW, _DIMP), jnp.float32),
        pltpu.VMEM((_BPW, _DIMP), jnp.float32),
        pltpu.VMEM((_BPW,), jnp.float32),
        pltpu.SemaphoreType.DMA,
    ],
    compiler_params=pltpu.CompilerParams(
        use_tc_tiling_on_sc=False, needs_layout_passes=False),
)
def _sc_dot(iw_hbm, ic_hbm, words_hbm, ctx_hbm, out_hbm,
            iw_v, ic_v, wrows, crows, outv, sem):
    wid = lax.axis_index("s") * _NC + lax.axis_index("c")
    rbase = wid * _NCH

    pltpu.sync_copy(iw_hbm.at[pl.ds(rbase, _NCH)], iw_v)
    pltpu.sync_copy(ic_hbm.at[pl.ds(rbase, _NCH)], ic_v)

    copies = []
    for j in range(_NCH):
        dst = pl.ds(j * _CH, _CH)
        copies.append(pltpu.async_copy(words_hbm.at[iw_v.at[j]], wrows.at[dst], sem))
        copies.append(pltpu.async_copy(ctx_hbm.at[ic_v.at[j]], crows.at[dst], sem))
    for cp in copies:
        cp.wait()

    def group(g, carry):
        rows = g * _L + lax.iota(jnp.int32, _L)
        acc = jnp.zeros((_L,), jnp.float32)
        for d in range(_DIM):
            col = jnp.full((_L,), d, jnp.int32)
            w = plsc.load_gather(wrows, [rows, col])
            c = plsc.load_gather(crows, [rows, col])
            acc = acc + w * c
        outv[pl.ds(g * _L, _L)] = acc
        return carry

    lax.fori_loop(0, _BPW // _L, group, 0)

    pltpu.sync_copy(outv, out_hbm.at[pl.ds(wid * _BPW, _BPW)])


def kernel(i_w, i_c, words, contexts):
    iw2 = i_w.astype(jnp.int32).reshape(_BATCH // _CH, _CH)
    ic2 = i_c.astype(jnp.int32).reshape(_BATCH // _CH, _CH)
    wp = jnp.pad(words, ((0, 0), (0, _DIMP - _DIM)))
    cp = jnp.pad(contexts, ((0, 0), (0, _DIMP - _DIM)))
    out = _sc_dot(iw2, ic2, wp, cp)
    return out.reshape(_BATCH, 1, 1)
